# initial kernel scaffold (unmeasured)
import jax
import jax.numpy as jnp
from jax import lax
from jax.experimental import pallas as pl
from jax.experimental.pallas import tpu as pltpu

N_DEV = 32
N_EXP = 128
E_PER = 4
CAP = 409


def kernel(x, router_W, route_idx, expert_W):
    del router_W
    n_tok, d = x.shape
    e_per, _, h = expert_W.shape

    e = route_idx[:, 0]
    oh = (e[:, None] == jnp.arange(N_EXP, dtype=jnp.int32)[None, :]).astype(
        jnp.int32
    )
    hist = oh.sum(axis=0, keepdims=True).astype(jnp.int32)
    rank = ((jnp.cumsum(oh, axis=0) - oh) * oh).sum(axis=1)
    rank = rank[:, None].astype(jnp.float32)

    x_bf = x.astype(jnp.bfloat16)
    w_bf = expert_W.astype(jnp.bfloat16)

    def body(x_ref, route_ref, w_ref, hist_ref, rank_ref, out_ref,
             w_comm, h_comm, hists, w_send, w_recv, h_send, h_recv):
        my = lax.axis_index("i")
        left = lax.rem(my + N_DEV - 1, N_DEV)
        right = lax.rem(my + 1, N_DEV)

        barrier = pltpu.get_barrier_semaphore()
        for nbr in (left, right):
            pl.semaphore_signal(barrier, inc=1, device_id=(nbr,),
                                device_id_type=pl.DeviceIdType.MESH)
        pl.semaphore_wait(barrier, 2)

        def block_acc(origin, w_block):
            acc = jnp.zeros((n_tok, h), jnp.float32)
            for k in range(E_PER):
                sel = (route_ref[...] == origin * E_PER + k).astype(jnp.bfloat16)
                acc = acc + jnp.dot(x_ref[...] * sel, w_block[k],
                                    preferred_element_type=jnp.float32)
            return acc

        w_comm[0] = w_ref[...]
        h_comm[0] = hist_ref[...]
        hists[pl.ds(my, 1), :] = hist_ref[...]
        out_ref[...] = block_acc(my, [w_ref[k] for k in range(E_PER)])

        for hop in range(N_DEV - 1):
            s, r = hop % 2, (hop + 1) % 2
            w_rdma = pltpu.make_async_remote_copy(
                src_ref=w_comm.at[s], dst_ref=w_comm.at[r],
                send_sem=w_send.at[s], recv_sem=w_recv.at[r],
                device_id=(right,), device_id_type=pl.DeviceIdType.MESH)
            h_rdma = pltpu.make_async_remote_copy(
                src_ref=h_comm.at[s], dst_ref=h_comm.at[r],
                send_sem=h_send.at[s], recv_sem=h_recv.at[r],
                device_id=(right,), device_id_type=pl.DeviceIdType.MESH)
            w_rdma.start()
            h_rdma.start()
            w_rdma.wait()
            h_rdma.wait()
            origin = lax.rem(my + (N_DEV - 1 - hop), N_DEV)
            hists[pl.ds(origin, 1), :] = h_comm[r]
            out_ref[...] += block_acc(
                origin, [w_comm[r, k] for k in range(E_PER)])

        row = lax.broadcasted_iota(jnp.int32, (N_DEV, N_EXP), 0)
        offsets = jnp.sum(jnp.where(row < my, hists[...], 0), axis=0,
                          keepdims=True).astype(jnp.float32)
        col = lax.broadcasted_iota(jnp.int32, (n_tok, N_EXP), 1)
        oh_tok = (route_ref[...] == col).astype(jnp.float32)
        offs_tok = jnp.sum(oh_tok * offsets, axis=1, keepdims=True)
        keep = ((rank_ref[...] + offs_tok) < CAP).astype(jnp.float32)
        out_ref[...] = out_ref[...] * keep

    params_cls = getattr(pltpu, "CompilerParams", None) or pltpu.TPUCompilerParams
    return pl.pallas_call(
        body,
        out_shape=jax.ShapeDtypeStruct((n_tok, h), jnp.float32),
        in_specs=[pl.BlockSpec(memory_space=pltpu.VMEM)] * 5,
        out_specs=pl.BlockSpec(memory_space=pltpu.VMEM),
        scratch_shapes=[
            pltpu.VMEM((2, e_per, d, h), jnp.bfloat16),
            pltpu.VMEM((2, 1, N_EXP), jnp.int32),
            pltpu.VMEM((N_DEV, N_EXP), jnp.int32),
            pltpu.SemaphoreType.DMA((2,)),
            pltpu.SemaphoreType.DMA((2,)),
            pltpu.SemaphoreType.DMA((2,)),
            pltpu.SemaphoreType.DMA((2,)),
        ],
        compiler_params=params_cls(collective_id=0),
    )(x_bf, route_idx, w_bf, hist, rank)


# baseline (device time: 1806034 ns/iter reference)
import jax
import jax.numpy as jnp
from jax import lax
from jax.experimental import pallas as pl
from jax.experimental.pallas import tpu as pltpu

N_DEV = 32
N_EXP = 128
E_PER = 4
CAP = 409


def kernel(x, router_W, route_idx, expert_W):
    del router_W
    n_tok, d = x.shape
    e_per, _, h = expert_W.shape

    e = route_idx[:, 0]
    oh = (e[:, None] == jnp.arange(N_EXP, dtype=jnp.int32)[None, :]).astype(
        jnp.int32
    )
    hist = oh.sum(axis=0, keepdims=True).astype(jnp.int32)
    rank = ((jnp.cumsum(oh, axis=0) - oh) * oh).sum(axis=1)
    rank = rank[:, None].astype(jnp.float32)

    x_bf = x.astype(jnp.bfloat16)
    w_bf = expert_W.astype(jnp.bfloat16)

    def body(x_ref, route_ref, w_ref, hist_ref, rank_ref, out_ref,
             w_comm, h_comm, hists, w_send, w_recv, h_send, h_recv):
        my = lax.axis_index("i")
        left = lax.rem(my + N_DEV - 1, N_DEV)
        right = lax.rem(my + 1, N_DEV)

        barrier = pltpu.get_barrier_semaphore()
        for nbr in (left, right):
            pl.semaphore_signal(barrier, inc=1, device_id=(nbr,),
                                device_id_type=pl.DeviceIdType.MESH)
        pl.semaphore_wait(barrier, 2)

        def block_acc(origin, w_flat):
            parts = [
                x_ref[...]
                * (route_ref[...] == origin * E_PER + k).astype(jnp.bfloat16)
                for k in range(E_PER)
            ]
            xm = jnp.concatenate(parts, axis=1)
            return jnp.dot(xm, w_flat, preferred_element_type=jnp.float32)

        def do_hop(hop, s, r):
            w_rdma = pltpu.make_async_remote_copy(
                src_ref=w_comm.at[s], dst_ref=w_comm.at[r],
                send_sem=w_send.at[s], recv_sem=w_recv.at[r],
                device_id=(right,), device_id_type=pl.DeviceIdType.MESH)
            h_rdma = pltpu.make_async_remote_copy(
                src_ref=h_comm.at[s], dst_ref=h_comm.at[r],
                send_sem=h_send.at[s], recv_sem=h_recv.at[r],
                device_id=(right,), device_id_type=pl.DeviceIdType.MESH)
            w_rdma.start()
            h_rdma.start()
            w_rdma.wait()
            h_rdma.wait()
            origin = lax.rem(my - hop - 1 + 2 * N_DEV, N_DEV)
            hists[pl.ds(origin, 1), :] = h_comm[r]
            out_ref[...] += block_acc(
                origin, jnp.reshape(w_comm[r], (E_PER * d, h)))

        w_comm[0] = w_ref[...]
        h_comm[0] = hist_ref[...]
        hists[pl.ds(my, 1), :] = hist_ref[...]
        out_ref[...] = block_acc(my, jnp.reshape(w_ref[...], (E_PER * d, h)))

        def pair(i, carry):
            do_hop(2 * i, 0, 1)
            do_hop(2 * i + 1, 1, 0)
            return carry

        lax.fori_loop(0, (N_DEV - 2) // 2, pair, 0)
        do_hop(N_DEV - 2, 0, 1)

        row = lax.broadcasted_iota(jnp.int32, (N_DEV, N_EXP), 0)
        offsets = jnp.sum(jnp.where(row < my, hists[...], 0), axis=0,
                          keepdims=True).astype(jnp.float32)
        col = lax.broadcasted_iota(jnp.int32, (n_tok, N_EXP), 1)
        oh_tok = (route_ref[...] == col).astype(jnp.float32)
        offs_tok = jnp.sum(oh_tok * offsets, axis=1, keepdims=True)
        keep = ((rank_ref[...] + offs_tok) < CAP).astype(jnp.float32)
        out_ref[...] = out_ref[...] * keep

    params_cls = getattr(pltpu, "CompilerParams", None) or pltpu.TPUCompilerParams
    return pl.pallas_call(
        body,
        out_shape=jax.ShapeDtypeStruct((n_tok, h), jnp.float32),
        in_specs=[pl.BlockSpec(memory_space=pltpu.VMEM)] * 5,
        out_specs=pl.BlockSpec(memory_space=pltpu.VMEM),
        scratch_shapes=[
            pltpu.VMEM((2, e_per, d, h), jnp.bfloat16),
            pltpu.VMEM((2, 1, N_EXP), jnp.int32),
            pltpu.VMEM((N_DEV, N_EXP), jnp.int32),
            pltpu.SemaphoreType.DMA((2,)),
            pltpu.SemaphoreType.DMA((2,)),
            pltpu.SemaphoreType.DMA((2,)),
            pltpu.SemaphoreType.DMA((2,)),
        ],
        compiler_params=params_cls(collective_id=0,
                                   vmem_limit_bytes=56 * 2**20),
    )(x_bf, route_idx, w_bf, hist, rank)


# device time: 1496791 ns/iter; 1.2066x vs baseline; 1.2066x over previous
import jax
import jax.numpy as jnp
from jax import lax
from jax.experimental import pallas as pl
from jax.experimental.pallas import tpu as pltpu

N_DEV = 32
N_EXP = 128
E_PER = 4
CAP = 409


def kernel(x, router_W, route_idx, expert_W):
    del router_W
    n_tok, d = x.shape
    e_per, _, h = expert_W.shape

    e = route_idx[:, 0]
    oh = (e[:, None] == jnp.arange(N_EXP, dtype=jnp.int32)[None, :]).astype(
        jnp.int32
    )
    hist = oh.sum(axis=0, keepdims=True).astype(jnp.int32)
    rank = ((jnp.cumsum(oh, axis=0) - oh) * oh).sum(axis=1)
    rank = rank[:, None].astype(jnp.float32)

    x_bf = x.astype(jnp.bfloat16)
    w_bf = expert_W.astype(jnp.bfloat16)

    def body(x_ref, route_ref, w_ref, hist_ref, rank_ref, out_ref,
             w_comm, h_comm, hists, w_send, w_recv, h_send, h_recv):
        my = lax.axis_index("i")
        left = lax.rem(my + N_DEV - 1, N_DEV)
        right = lax.rem(my + 1, N_DEV)

        barrier = pltpu.get_barrier_semaphore()
        for nbr in (left, right):
            pl.semaphore_signal(barrier, inc=1, device_id=(nbr,),
                                device_id_type=pl.DeviceIdType.MESH)
        pl.semaphore_wait(barrier, 2)

        def block_acc(origin, w_flat):
            parts = [
                x_ref[...]
                * (route_ref[...] == origin * E_PER + k).astype(jnp.bfloat16)
                for k in range(E_PER)
            ]
            xm = jnp.concatenate(parts, axis=1)
            return jnp.dot(xm, w_flat, preferred_element_type=jnp.float32)

        def do_hop(hop, s, r):
            w_rdma = pltpu.make_async_remote_copy(
                src_ref=w_comm.at[s], dst_ref=w_comm.at[r],
                send_sem=w_send.at[s], recv_sem=w_recv.at[r],
                device_id=(right,), device_id_type=pl.DeviceIdType.MESH)
            h_rdma = pltpu.make_async_remote_copy(
                src_ref=h_comm.at[s], dst_ref=h_comm.at[r],
                send_sem=h_send.at[s], recv_sem=h_recv.at[r],
                device_id=(right,), device_id_type=pl.DeviceIdType.MESH)
            w_rdma.start()
            h_rdma.start()
            origin_prev = lax.rem(my - hop + 2 * N_DEV, N_DEV)
            out_ref[...] += block_acc(
                origin_prev, jnp.reshape(w_comm[s], (E_PER * d, h)))
            w_rdma.wait()
            h_rdma.wait()
            origin = lax.rem(my - hop - 1 + 2 * N_DEV, N_DEV)
            hists[pl.ds(origin, 1), :] = h_comm[r]

        w_comm[0] = w_ref[...]
        h_comm[0] = hist_ref[...]
        hists[pl.ds(my, 1), :] = hist_ref[...]
        out_ref[...] = jnp.zeros((n_tok, h), jnp.float32)

        def pair(i, carry):
            do_hop(2 * i, 0, 1)
            do_hop(2 * i + 1, 1, 0)
            return carry

        lax.fori_loop(0, (N_DEV - 2) // 2, pair, 0)
        do_hop(N_DEV - 2, 0, 1)
        out_ref[...] += block_acc(
            lax.rem(my + 1, N_DEV), jnp.reshape(w_comm[1], (E_PER * d, h)))

        row = lax.broadcasted_iota(jnp.int32, (N_DEV, N_EXP), 0)
        offsets = jnp.sum(jnp.where(row < my, hists[...], 0), axis=0,
                          keepdims=True).astype(jnp.float32)
        col = lax.broadcasted_iota(jnp.int32, (n_tok, N_EXP), 1)
        oh_tok = (route_ref[...] == col).astype(jnp.float32)
        offs_tok = jnp.sum(oh_tok * offsets, axis=1, keepdims=True)
        keep = ((rank_ref[...] + offs_tok) < CAP).astype(jnp.float32)
        out_ref[...] = out_ref[...] * keep

    params_cls = getattr(pltpu, "CompilerParams", None) or pltpu.TPUCompilerParams
    return pl.pallas_call(
        body,
        out_shape=jax.ShapeDtypeStruct((n_tok, h), jnp.float32),
        in_specs=[pl.BlockSpec(memory_space=pltpu.VMEM)] * 5,
        out_specs=pl.BlockSpec(memory_space=pltpu.VMEM),
        scratch_shapes=[
            pltpu.VMEM((2, e_per, d, h), jnp.bfloat16),
            pltpu.VMEM((2, 1, N_EXP), jnp.int32),
            pltpu.VMEM((N_DEV, N_EXP), jnp.int32),
            pltpu.SemaphoreType.DMA((2,)),
            pltpu.SemaphoreType.DMA((2,)),
            pltpu.SemaphoreType.DMA((2,)),
            pltpu.SemaphoreType.DMA((2,)),
        ],
        compiler_params=params_cls(collective_id=0,
                                   vmem_limit_bytes=56 * 2**20),
    )(x_bf, route_idx, w_bf, hist, rank)
